# baseline (speedup 1.0000x reference)
import numpy as np

import jax
import jax.numpy as jnp
from jax.experimental import pallas as pl
from jax.experimental.pallas import tpu as pltpu
from jax.sharding import Mesh, PartitionSpec as P

_K = 4096
_N = 4096
_BM = 256
_BN = 2048
_S = 256
_NSTRIP = _BN // _S


def _matmul_kernel(x_ref, w_hbm, b_ref, o_ref, wbf, stage, sems):
    n = pl.program_id(0)
    m = pl.program_id(1)

    def start_strip(nblock, strip, buf):
        pltpu.make_async_copy(
            w_hbm.at[pl.ds(nblock * _BN + strip * _S, _S), :],
            stage.at[pl.ds(buf * _S, _S), :],
            sems.at[buf],
        ).start()

    def finish_strip(nblock, strip, buf):
        pltpu.make_async_copy(
            w_hbm.at[pl.ds(0, _S), :],
            stage.at[pl.ds(buf * _S, _S), :],
            sems.at[buf],
        ).wait()
        wbf[pl.ds(nblock * _BN + strip * _S, _S), :] = (
            stage[pl.ds(buf * _S, _S), :].astype(jnp.bfloat16))

    @pl.when(jnp.logical_and(n == 0, m == 0))
    def _load_first_block():
        start_strip(0, 0, 0)
        start_strip(0, 1, 1)
        for s in range(_NSTRIP):
            finish_strip(0, s, s % 2)
            if s + 2 < _NSTRIP:
                start_strip(0, s + 2, s % 2)

    @pl.when(jnp.logical_and(n == 0,
                             jnp.logical_and(m >= 3, m <= _NSTRIP + 2)))
    def _finish_prefetch():
        finish_strip(1, m - 3, (m - 3) % 2)

    @pl.when(jnp.logical_and(n == 0,
                             jnp.logical_and(m >= 1, m <= _NSTRIP)))
    def _start_prefetch():
        start_strip(1, m - 1, (m - 1) % 2)

    xb = x_ref[...].astype(jnp.bfloat16)
    wb = wbf[pl.ds(n * _BN, _BN), :]
    acc = jax.lax.dot_general(
        xb, wb, (((1,), (1,)), ((), ())),
        preferred_element_type=jnp.float32)
    o_ref[...] = acc + b_ref[...]


def _pallas_matmul(x2d, W, b2):
    m_tokens = x2d.shape[0]
    grid = (_N // _BN, m_tokens // _BM)
    return pl.pallas_call(
        _matmul_kernel,
        grid=grid,
        in_specs=[
            pl.BlockSpec((_BM, _K), lambda n, m: (m, 0)),
            pl.BlockSpec(memory_space=pl.ANY),
            pl.BlockSpec((1, _BN), lambda n, m: (0, n)),
        ],
        out_specs=pl.BlockSpec((_BM, _BN), lambda n, m: (m, n)),
        out_shape=jax.ShapeDtypeStruct((m_tokens, _N), jnp.float32),
        scratch_shapes=[
            pltpu.VMEM((2 * _BN, _K), jnp.bfloat16),
            pltpu.VMEM((2 * _S, _K), jnp.float32),
            pltpu.SemaphoreType.DMA((2,)),
        ],
        compiler_params=pltpu.CompilerParams(
            dimension_semantics=("arbitrary", "arbitrary"),
        ),
    )(x2d, W, b2)


def kernel(x, W, b):
    x_flat = x.reshape(-1, _K)
    b2 = b.reshape(1, _N)
    devs = jax.devices()
    if len(devs) >= 2 and all(d.platform == "tpu" for d in devs[:2]):
        mesh = Mesh(np.array(devs[:2]), ("i",))
        out = jax.shard_map(
            _pallas_matmul, mesh=mesh,
            in_specs=(P("i", None), P(None, None), P(None, None)),
            out_specs=P("i", None), check_vma=False,
        )(x_flat, W, b2)
    else:
        out = _pallas_matmul(x_flat, W, b2)
    return out.reshape(x.shape[0], x.shape[1], _N)


# speedup vs baseline: 2.1218x; 2.1218x over previous
import jax
import jax.numpy as jnp
from jax.experimental import pallas as pl
from jax.experimental.pallas import tpu as pltpu

_K = 4096
_N = 4096
_BM = 256
_BN = 2048
_S = 256
_NSTRIP = _BN // _S


def _matmul_kernel(x_ref, w_hbm, b_ref, o_ref, wbf, stage, sems):
    n = pl.program_id(0)
    m = pl.program_id(1)

    def start_strip(nblock, strip, buf):
        pltpu.make_async_copy(
            w_hbm.at[pl.ds(nblock * _BN + strip * _S, _S), :],
            stage.at[pl.ds(buf * _S, _S), :],
            sems.at[buf],
        ).start()

    def finish_strip(nblock, strip, buf):
        pltpu.make_async_copy(
            w_hbm.at[pl.ds(0, _S), :],
            stage.at[pl.ds(buf * _S, _S), :],
            sems.at[buf],
        ).wait()
        wbf[pl.ds(nblock * _BN + strip * _S, _S), :] = (
            stage[pl.ds(buf * _S, _S), :].astype(jnp.bfloat16))

    @pl.when(jnp.logical_and(n == 0, m == 0))
    def _load_first_block():
        start_strip(0, 0, 0)
        start_strip(0, 1, 1)
        for s in range(_NSTRIP):
            finish_strip(0, s, s % 2)
            if s + 2 < _NSTRIP:
                start_strip(0, s + 2, s % 2)

    @pl.when(jnp.logical_and(n == 0,
                             jnp.logical_and(m >= 3, m <= _NSTRIP + 2)))
    def _finish_prefetch():
        finish_strip(1, m - 3, (m - 3) % 2)

    @pl.when(jnp.logical_and(n == 0,
                             jnp.logical_and(m >= 1, m <= _NSTRIP)))
    def _start_prefetch():
        start_strip(1, m - 1, (m - 1) % 2)

    xb = x_ref[...].astype(jnp.bfloat16)
    wb = wbf[pl.ds(n * _BN, _BN), :]
    acc = jax.lax.dot_general(
        xb, wb, (((1,), (1,)), ((), ())),
        preferred_element_type=jnp.float32)
    o_ref[...] = acc + b_ref[...]


def _pallas_matmul(x2d, W, b2):
    m_tokens = x2d.shape[0]
    grid = (_N // _BN, m_tokens // _BM)
    return pl.pallas_call(
        _matmul_kernel,
        grid=grid,
        in_specs=[
            pl.BlockSpec((_BM, _K), lambda n, m: (m, 0)),
            pl.BlockSpec(memory_space=pl.ANY),
            pl.BlockSpec((1, _BN), lambda n, m: (0, n)),
        ],
        out_specs=pl.BlockSpec((_BM, _BN), lambda n, m: (m, n)),
        out_shape=jax.ShapeDtypeStruct((m_tokens, _N), jnp.float32),
        scratch_shapes=[
            pltpu.VMEM((2 * _BN, _K), jnp.bfloat16),
            pltpu.VMEM((2 * _S, _K), jnp.float32),
            pltpu.SemaphoreType.DMA((2,)),
        ],
        compiler_params=pltpu.CompilerParams(
            dimension_semantics=("arbitrary", "arbitrary"),
        ),
    )(x2d, W, b2)


def kernel(x, W, b):
    x_flat = x.reshape(-1, _K)
    b2 = b.reshape(1, _N)
    out = _pallas_matmul(x_flat, W, b2)
    return out.reshape(x.shape[0], x.shape[1], _N)


# speedup vs baseline: 2.1260x; 1.0020x over previous
import jax
import jax.numpy as jnp
from jax.experimental import pallas as pl
from jax.experimental.pallas import tpu as pltpu

_K = 4096
_N = 4096
_BM = 256
_BN = 2048
_S = 256
_NSTRIP = _BN // _S


def _matmul_kernel(x_ref, w_hbm, b_ref, o_ref, wbf, stage, sems):
    n = pl.program_id(0)
    m = pl.program_id(1)

    def start_strip(nblock, strip, buf):
        pltpu.make_async_copy(
            w_hbm.at[pl.ds(nblock * _BN + strip * _S, _S), :],
            stage.at[pl.ds(buf * _S, _S), :],
            sems.at[buf],
        ).start()

    def finish_strip(nblock, strip, buf):
        pltpu.make_async_copy(
            w_hbm.at[pl.ds(0, _S), :],
            stage.at[pl.ds(buf * _S, _S), :],
            sems.at[buf],
        ).wait()
        wbf[pl.ds(nblock * _BN + strip * _S, _S), :] = (
            stage[pl.ds(buf * _S, _S), :].astype(jnp.bfloat16))

    is_first = jnp.logical_and(n == 0, m == 0)

    @pl.when(is_first)
    def _load_first_block():
        start_strip(0, 0, 0)
        start_strip(0, 1, 1)
        xb = x_ref[...].astype(jnp.bfloat16)
        for s in range(_NSTRIP):
            finish_strip(0, s, s % 2)
            if s + 2 < _NSTRIP:
                start_strip(0, s + 2, s % 2)
            ws = wbf[pl.ds(s * _S, _S), :]
            acc = jax.lax.dot_general(
                xb, ws, (((1,), (1,)), ((), ())),
                preferred_element_type=jnp.float32)
            o_ref[:, s * _S:(s + 1) * _S] = (
                acc + b_ref[:, s * _S:(s + 1) * _S])

    @pl.when(jnp.logical_and(n == 0,
                             jnp.logical_and(m >= 3, m <= _NSTRIP + 2)))
    def _finish_prefetch():
        finish_strip(1, m - 3, (m - 3) % 2)

    @pl.when(jnp.logical_and(n == 0,
                             jnp.logical_and(m >= 1, m <= _NSTRIP)))
    def _start_prefetch():
        start_strip(1, m - 1, (m - 1) % 2)

    @pl.when(jnp.logical_not(is_first))
    def _compute():
        xb = x_ref[...].astype(jnp.bfloat16)
        wb = wbf[pl.ds(n * _BN, _BN), :]
        acc = jax.lax.dot_general(
            xb, wb, (((1,), (1,)), ((), ())),
            preferred_element_type=jnp.float32)
        o_ref[...] = acc + b_ref[...]


def _pallas_matmul(x2d, W, b2):
    m_tokens = x2d.shape[0]
    grid = (_N // _BN, m_tokens // _BM)
    return pl.pallas_call(
        _matmul_kernel,
        grid=grid,
        in_specs=[
            pl.BlockSpec((_BM, _K), lambda n, m: (m, 0)),
            pl.BlockSpec(memory_space=pl.ANY),
            pl.BlockSpec((1, _BN), lambda n, m: (0, n)),
        ],
        out_specs=pl.BlockSpec((_BM, _BN), lambda n, m: (m, n)),
        out_shape=jax.ShapeDtypeStruct((m_tokens, _N), jnp.float32),
        scratch_shapes=[
            pltpu.VMEM((2 * _BN, _K), jnp.bfloat16),
            pltpu.VMEM((2 * _S, _K), jnp.float32),
            pltpu.SemaphoreType.DMA((2,)),
        ],
        compiler_params=pltpu.CompilerParams(
            dimension_semantics=("arbitrary", "arbitrary"),
        ),
    )(x2d, W, b2)


def kernel(x, W, b):
    x_flat = x.reshape(-1, _K)
    b2 = b.reshape(1, _N)
    out = _pallas_matmul(x_flat, W, b2)
    return out.reshape(x.shape[0], x.shape[1], _N)
